# baseline (speedup 1.0000x reference)
import functools

import jax
import jax.numpy as jnp
from jax import lax
from jax.experimental import pallas as pl
from jax.experimental.pallas import tpu as pltpu
from jax.experimental.pallas import tpu_sc as plsc

N_NODES = 10000
N_EDGES = 320000
F = 128
NC = 2
NS = 16
EC = 32
CPS = 32
NSLOT = 8
SCT = 10
CHT = CPS * SCT
E_PAD = NC * NS * CHT * EC
NP = 10240
ROWS_PT = NP // NS
BR = 2000
BRF = 640


def _tc_matmul_body(h_ref, w_ref, norm_ref, out_ref):
    out_ref[...] = jnp.dot(h_ref[...], w_ref[...],
                           preferred_element_type=jnp.float32) * norm_ref[...]


def _tc_matmul(h, weight, norm):
    grid = (N_NODES // BR,)
    return pl.pallas_call(
        _tc_matmul_body,
        grid=grid,
        in_specs=[
            pl.BlockSpec((BR, F), lambda r: (r, 0)),
            pl.BlockSpec((F, F), lambda r: (0, 0)),
            pl.BlockSpec((BR, 1), lambda r: (r, 0)),
        ],
        out_specs=pl.BlockSpec((BR, F), lambda r: (r, 0)),
        out_shape=jax.ShapeDtypeStruct((N_NODES, F), jnp.float32),
    )(h, weight, norm)


def _tc_final_body(a0_ref, a1_ref, norm_ref, bias_ref, out_ref):
    agg = a0_ref[...] + a1_ref[...]
    out_ref[...] = agg * norm_ref[...] + bias_ref[...]


def _tc_final(parts, norm, bias2):
    grid = (NP // BRF,)
    nb = NP // BRF
    return pl.pallas_call(
        _tc_final_body,
        grid=grid,
        in_specs=[
            pl.BlockSpec((BRF, F), lambda r: (r, 0)),
            pl.BlockSpec((BRF, F), lambda r: (nb + r, 0)),
            pl.BlockSpec((BRF, 1), lambda r: (r, 0)),
            pl.BlockSpec((1, F), lambda r: (0, 0)),
        ],
        out_specs=pl.BlockSpec((BRF, F), lambda r: (r, 0)),
        out_shape=jax.ShapeDtypeStruct((N_NODES, F), jnp.float32),
    )(parts, parts, norm, bias2)


def _sc_agg_body(table_hbm, src_hbm, dst_hbm, zeros_hbm, out_hbm,
                 ibs0, ibs1, ibd0, ibd1, *rest):
    rows = rest[:NSLOT]
    acc_sh = rest[NSLOT]
    isem = rest[NSLOT + 1:NSLOT + 3]
    gsem = rest[NSLOT + 3:2 * NSLOT + 3]
    csem = rest[2 * NSLOT + 3:]
    c = lax.axis_index("c")
    s = lax.axis_index("s")
    wid = c * NS + s
    base = wid * CHT
    ibs = (ibs0, ibs1)
    ibd = (ibd0, ibd1)

    zd = pltpu.async_copy(zeros_hbm.at[pl.ds(s * ROWS_PT, ROWS_PT)],
                          acc_sh.at[pl.ds(s * ROWS_PT, ROWS_PT)], gsem[0])
    for par in range(2):
        pltpu.async_copy(src_hbm.at[pl.ds(base + par * CPS, CPS)],
                         ibs[par], isem[par])
        pltpu.async_copy(dst_hbm.at[pl.ds(base + par * CPS, CPS)],
                         ibd[par], isem[par])
    zd.wait()
    plsc.subcore_barrier()

    def two_supers(t, carry):
        for par in range(2):
            sc_i = 2 * t + par
            pltpu.make_async_copy(src_hbm.at[pl.ds(base, CPS)],
                                  ibs[par], isem[par]).wait()
            pltpu.make_async_copy(dst_hbm.at[pl.ds(base, CPS)],
                                  ibd[par], isem[par]).wait()
            gd = {}
            cd = {}
            for b in range(NSLOT):
                gd[b] = pltpu.async_copy(table_hbm.at[ibs[par].at[b]],
                                         rows[b], gsem[b])
            for b in range(CPS):
                gd[b].wait()
                cd[b] = pltpu.async_copy(rows[b % NSLOT],
                                         acc_sh.at[ibd[par].at[b]],
                                         csem[b % NSLOT], add=True)
                if b + NSLOT < CPS:
                    cd[b].wait()
                    gd[b + NSLOT] = pltpu.async_copy(
                        table_hbm.at[ibs[par].at[b + NSLOT]],
                        rows[b % NSLOT], gsem[b % NSLOT])
            for b in range(CPS - NSLOT, CPS):
                cd[b].wait()
            nxt = base + jnp.minimum(sc_i + 2, SCT - 1) * CPS
            pltpu.async_copy(src_hbm.at[pl.ds(nxt, CPS)], ibs[par], isem[par])
            pltpu.async_copy(dst_hbm.at[pl.ds(nxt, CPS)], ibd[par], isem[par])
        return carry

    lax.fori_loop(0, SCT // 2, two_supers, 0)
    for par in range(2):
        pltpu.make_async_copy(src_hbm.at[pl.ds(base, CPS)],
                              ibs[par], isem[par]).wait()
        pltpu.make_async_copy(dst_hbm.at[pl.ds(base, CPS)],
                              ibd[par], isem[par]).wait()
    plsc.subcore_barrier()

    pltpu.sync_copy(acc_sh.at[pl.ds(s * ROWS_PT, ROWS_PT)],
                    out_hbm.at[pl.ds(c * NP + s * ROWS_PT, ROWS_PT)])


_sc_agg = functools.partial(
    pl.kernel,
    mesh=plsc.VectorSubcoreMesh(core_axis_name="c", subcore_axis_name="s",
                                num_cores=NC, num_subcores=NS),
    out_type=jax.ShapeDtypeStruct((NC * NP, F), jnp.float32),
    scratch_types=[
        pltpu.VMEM((CPS, EC), jnp.int32),
        pltpu.VMEM((CPS, EC), jnp.int32),
        pltpu.VMEM((CPS, EC), jnp.int32),
        pltpu.VMEM((CPS, EC), jnp.int32),
    ] + [pltpu.VMEM((EC, F), jnp.float32) for _ in range(NSLOT)] + [
        pltpu.VMEM_SHARED((NP, F), jnp.float32),
    ] + [pltpu.SemaphoreType.DMA for _ in range(2 + 2 * NSLOT)],
)(_sc_agg_body)


def kernel(h, edge_index, norm, weight, bias):
    src = edge_index[0].astype(jnp.int32)
    dst = edge_index[1].astype(jnp.int32)
    npad = E_PAD - N_EDGES
    pad_iota = lax.iota(jnp.int32, npad)
    src_p = jnp.concatenate([src, pad_iota % N_NODES]).reshape(-1, EC)
    dst_p = jnp.concatenate([dst, N_NODES + pad_iota % (NP - N_NODES)]
                            ).reshape(-1, EC)
    hw = _tc_matmul(h, weight, norm)
    zeros = jnp.zeros((NP, F), jnp.float32)
    parts = _sc_agg(hw, src_p, dst_p, zeros)
    return _tc_final(parts, norm, bias.reshape(1, F))


# speedup vs baseline: 1.0803x; 1.0803x over previous
import functools

import jax
import jax.numpy as jnp
from jax import lax
from jax.experimental import pallas as pl
from jax.experimental.pallas import tpu as pltpu
from jax.experimental.pallas import tpu_sc as plsc

N_NODES = 10000
N_EDGES = 320000
F = 128
NC = 2
NS = 16
EC = 32
CPS = 32
NSLOT = 8
SCT = 10
CHT = CPS * SCT
E_PAD = NC * NS * CHT * EC
NP = 10240
ROWS_PT = NP // NS
BRF = 2048


def _tc_final_body(a0_ref, a1_ref, w_ref, norm_ref, bias_ref, out_ref):
    agg = a0_ref[...] + a1_ref[...]
    mm = jnp.dot(agg, w_ref[...], preferred_element_type=jnp.float32)
    out_ref[...] = mm * norm_ref[...] + bias_ref[...]


def _tc_final(parts, weight, norm, bias2):
    nb = NP // BRF
    return pl.pallas_call(
        _tc_final_body,
        grid=(nb,),
        in_specs=[
            pl.BlockSpec((BRF, F), lambda r: (r, 0)),
            pl.BlockSpec((BRF, F), lambda r: (nb + r, 0)),
            pl.BlockSpec((F, F), lambda r: (0, 0)),
            pl.BlockSpec((BRF, 1), lambda r: (r, 0)),
            pl.BlockSpec((1, F), lambda r: (0, 0)),
        ],
        out_specs=pl.BlockSpec((BRF, F), lambda r: (r, 0)),
        out_shape=jax.ShapeDtypeStruct((N_NODES, F), jnp.float32),
    )(parts, parts, weight, norm, bias2)


def _sc_agg_body(table_hbm, sd_hbm, zeros_hbm, out_hbm,
                 ibs0, ibs1, ibd0, ibd1, *rest):
    rows = rest[:NSLOT]
    acc_sh = rest[NSLOT]
    isem = rest[NSLOT + 1:NSLOT + 3]
    gsem = rest[NSLOT + 3:2 * NSLOT + 3]
    csem = rest[2 * NSLOT + 3:]
    c = lax.axis_index("c")
    s = lax.axis_index("s")
    wid = c * NS + s
    base = wid * CHT
    ibs = (ibs0, ibs1)
    ibd = (ibd0, ibd1)

    zd = pltpu.async_copy(zeros_hbm.at[pl.ds(s * ROWS_PT, ROWS_PT)],
                          acc_sh.at[pl.ds(s * ROWS_PT, ROWS_PT)], gsem[0])
    for par in range(2):
        pltpu.async_copy(sd_hbm.at[0, pl.ds(base + par * CPS, CPS)],
                         ibs[par], isem[par])
        pltpu.async_copy(sd_hbm.at[1, pl.ds(base + par * CPS, CPS)],
                         ibd[par], isem[par])
    zd.wait()
    plsc.subcore_barrier()

    def two_supers(t, carry):
        for par in range(2):
            sc_i = 2 * t + par
            pltpu.make_async_copy(sd_hbm.at[0, pl.ds(base, CPS)],
                                  ibs[par], isem[par]).wait()
            pltpu.make_async_copy(sd_hbm.at[1, pl.ds(base, CPS)],
                                  ibd[par], isem[par]).wait()
            gd = {}
            cd = {}
            for b in range(NSLOT):
                gd[b] = pltpu.async_copy(table_hbm.at[ibs[par].at[b]],
                                         rows[b], gsem[b])
            for b in range(CPS):
                gd[b].wait()
                cd[b] = pltpu.async_copy(rows[b % NSLOT],
                                         acc_sh.at[ibd[par].at[b]],
                                         csem[b % NSLOT], add=True)
                if b + NSLOT < CPS:
                    cd[b].wait()
                    gd[b + NSLOT] = pltpu.async_copy(
                        table_hbm.at[ibs[par].at[b + NSLOT]],
                        rows[b % NSLOT], gsem[b % NSLOT])
            for b in range(CPS - NSLOT, CPS):
                cd[b].wait()
            nxt = base + jnp.minimum(sc_i + 2, SCT - 1) * CPS
            pltpu.async_copy(sd_hbm.at[0, pl.ds(nxt, CPS)], ibs[par],
                             isem[par])
            pltpu.async_copy(sd_hbm.at[1, pl.ds(nxt, CPS)], ibd[par],
                             isem[par])
        return carry

    lax.fori_loop(0, SCT // 2, two_supers, 0)
    for par in range(2):
        pltpu.make_async_copy(sd_hbm.at[0, pl.ds(base, CPS)],
                              ibs[par], isem[par]).wait()
        pltpu.make_async_copy(sd_hbm.at[1, pl.ds(base, CPS)],
                              ibd[par], isem[par]).wait()
    plsc.subcore_barrier()

    pltpu.sync_copy(acc_sh.at[pl.ds(s * ROWS_PT, ROWS_PT)],
                    out_hbm.at[pl.ds(c * NP + s * ROWS_PT, ROWS_PT)])


_sc_agg = functools.partial(
    pl.kernel,
    mesh=plsc.VectorSubcoreMesh(core_axis_name="c", subcore_axis_name="s",
                                num_cores=NC, num_subcores=NS),
    out_type=jax.ShapeDtypeStruct((NC * NP, F), jnp.float32),
    scratch_types=[
        pltpu.VMEM((CPS, EC), jnp.int32),
        pltpu.VMEM((CPS, EC), jnp.int32),
        pltpu.VMEM((CPS, EC), jnp.int32),
        pltpu.VMEM((CPS, EC), jnp.int32),
    ] + [pltpu.VMEM((EC, F), jnp.float32) for _ in range(NSLOT)] + [
        pltpu.VMEM_SHARED((NP, F), jnp.float32),
    ] + [pltpu.SemaphoreType.DMA for _ in range(2 + 2 * NSLOT)],
)(_sc_agg_body)


def kernel(h, edge_index, norm, weight, bias):
    hs = h * norm
    npad = E_PAD - N_EDGES
    pad_iota = lax.iota(jnp.int32, npad)
    pads = jnp.stack([pad_iota % N_NODES,
                      N_NODES + pad_iota % (NP - N_NODES)])
    sdp = jnp.concatenate([edge_index.astype(jnp.int32), pads],
                          axis=1).reshape(2, -1, EC)
    zeros = jnp.zeros((NP, F), jnp.float32)
    parts = _sc_agg(hs, sdp, zeros)
    return _tc_final(parts, weight, norm, bias.reshape(1, F))


# speedup vs baseline: 1.1142x; 1.0313x over previous
import functools

import jax
import jax.numpy as jnp
from jax import lax
from jax.experimental import pallas as pl
from jax.experimental.pallas import tpu as pltpu
from jax.experimental.pallas import tpu_sc as plsc

N_NODES = 10000
N_EDGES = 320000
F = 128
NC = 2
NS = 16
EC = 32
CPS = 32
NSLOT = 8
SCT = 10
CHT = CPS * SCT
E_PAD = NC * NS * CHT * EC
NP = 10240
ROWS_PT = NP // NS
BRF = 2048


def _tc_final_body(a0_ref, a1_ref, w_ref, norm_ref, bias_ref, out_ref):
    agg = a0_ref[...] + a1_ref[...]
    mm = jnp.dot(agg, w_ref[...], preferred_element_type=jnp.float32)
    out_ref[...] = mm * norm_ref[...] + bias_ref[...]


def _tc_final(parts, weight, norm, bias2):
    nb = NP // BRF
    return pl.pallas_call(
        _tc_final_body,
        grid=(nb,),
        in_specs=[
            pl.BlockSpec((BRF, F), lambda r: (r, 0)),
            pl.BlockSpec((BRF, F), lambda r: (nb + r, 0)),
            pl.BlockSpec((F, F), lambda r: (0, 0)),
            pl.BlockSpec((BRF, 1), lambda r: (r, 0)),
            pl.BlockSpec((1, F), lambda r: (0, 0)),
        ],
        out_specs=pl.BlockSpec((BRF, F), lambda r: (r, 0)),
        out_shape=jax.ShapeDtypeStruct((N_NODES, F), jnp.float32),
    )(parts, parts, weight, norm, bias2)


def _sc_agg_body(table_hbm, sd_hbm, out_hbm,
                 ibs0, ibs1, ibd0, ibd1, *rest):
    rows = rest[:NSLOT]
    acc_sh = rest[NSLOT]
    isem = rest[NSLOT + 1:NSLOT + 3]
    gsem = rest[NSLOT + 3:2 * NSLOT + 3]
    csem = rest[2 * NSLOT + 3:]
    c = lax.axis_index("c")
    s = lax.axis_index("s")
    wid = c * NS + s
    base = wid * CHT
    ibs = (ibs0, ibs1)
    ibd = (ibd0, ibd1)

    for par in range(2):
        pltpu.async_copy(sd_hbm.at[0, pl.ds(base + par * CPS, CPS)],
                         ibs[par], isem[par])
        pltpu.async_copy(sd_hbm.at[1, pl.ds(base + par * CPS, CPS)],
                         ibd[par], isem[par])

    zvec = jnp.zeros((16,), jnp.float32)

    def zrow(i, carry):
        for t in range(F // 16):
            rows[0][i, pl.ds(t * 16, 16)] = zvec
        return carry

    lax.fori_loop(0, EC, zrow, 0)
    for q in range(ROWS_PT // EC):
        pltpu.sync_copy(rows[0],
                        acc_sh.at[pl.ds(s * ROWS_PT + q * EC, EC)])
    plsc.subcore_barrier()

    def two_supers(t, carry):
        for par in range(2):
            sc_i = 2 * t + par
            pltpu.make_async_copy(sd_hbm.at[0, pl.ds(base, CPS)],
                                  ibs[par], isem[par]).wait()
            pltpu.make_async_copy(sd_hbm.at[1, pl.ds(base, CPS)],
                                  ibd[par], isem[par]).wait()
            gd = {}
            cd = {}
            for b in range(NSLOT):
                gd[b] = pltpu.async_copy(table_hbm.at[ibs[par].at[b]],
                                         rows[b], gsem[b])
            for b in range(CPS):
                gd[b].wait()
                cd[b] = pltpu.async_copy(rows[b % NSLOT],
                                         acc_sh.at[ibd[par].at[b]],
                                         csem[b % NSLOT], add=True)
                if b + NSLOT < CPS:
                    cd[b].wait()
                    gd[b + NSLOT] = pltpu.async_copy(
                        table_hbm.at[ibs[par].at[b + NSLOT]],
                        rows[b % NSLOT], gsem[b % NSLOT])
            for b in range(CPS - NSLOT, CPS):
                cd[b].wait()
            nxt = base + jnp.minimum(sc_i + 2, SCT - 1) * CPS
            pltpu.async_copy(sd_hbm.at[0, pl.ds(nxt, CPS)], ibs[par],
                             isem[par])
            pltpu.async_copy(sd_hbm.at[1, pl.ds(nxt, CPS)], ibd[par],
                             isem[par])
        return carry

    lax.fori_loop(0, SCT // 2, two_supers, 0)
    for par in range(2):
        pltpu.make_async_copy(sd_hbm.at[0, pl.ds(base, CPS)],
                              ibs[par], isem[par]).wait()
        pltpu.make_async_copy(sd_hbm.at[1, pl.ds(base, CPS)],
                              ibd[par], isem[par]).wait()
    plsc.subcore_barrier()

    pltpu.sync_copy(acc_sh.at[pl.ds(s * ROWS_PT, ROWS_PT)],
                    out_hbm.at[pl.ds(c * NP + s * ROWS_PT, ROWS_PT)])


_sc_agg = functools.partial(
    pl.kernel,
    mesh=plsc.VectorSubcoreMesh(core_axis_name="c", subcore_axis_name="s",
                                num_cores=NC, num_subcores=NS),
    out_type=jax.ShapeDtypeStruct((NC * NP, F), jnp.float32),
    scratch_types=[
        pltpu.VMEM((CPS, EC), jnp.int32),
        pltpu.VMEM((CPS, EC), jnp.int32),
        pltpu.VMEM((CPS, EC), jnp.int32),
        pltpu.VMEM((CPS, EC), jnp.int32),
    ] + [pltpu.VMEM((EC, F), jnp.float32) for _ in range(NSLOT)] + [
        pltpu.VMEM_SHARED((NP, F), jnp.float32),
    ] + [pltpu.SemaphoreType.DMA for _ in range(2 + 2 * NSLOT)],
)(_sc_agg_body)


def kernel(h, edge_index, norm, weight, bias):
    hs = h * norm
    npad = E_PAD - N_EDGES
    pad_iota = lax.iota(jnp.int32, npad)
    pads = jnp.stack([pad_iota % N_NODES,
                      N_NODES + pad_iota % (NP - N_NODES)])
    sdp = jnp.concatenate([edge_index.astype(jnp.int32), pads],
                          axis=1).reshape(2, -1, EC)
    parts = _sc_agg(hs, sdp)
    return _tc_final(parts, weight, norm, bias.reshape(1, F))
